# Initial kernel scaffold; baseline (speedup 1.0000x reference)
import jax
import jax.numpy as jnp
from jax.experimental import pallas as pl


def kernel(x_station, x_machine, x_robot, x_job, edge_can_load, edge_loaded, edge_will_execute, edge_execute, edge_hold, Wl_can_load, bl_can_load, Wr_can_load, Wl_loaded, bl_loaded, Wr_loaded, Wl_will_execute, bl_will_execute, Wr_will_execute, Wl_execute, bl_execute, Wr_execute, Wl_hold, bl_hold, Wr_hold):
    raise NotImplementedError("write your pallas kernel here")


# speedup vs baseline: 4.1556x; 4.1556x over previous
import functools

import jax
import jax.numpy as jnp
from jax import lax
from jax.experimental import pallas as pl
from jax.experimental.pallas import tpu as pltpu
from jax.experimental.pallas import tpu_sc as plsc

N = 10000
E = 320000
D = 128
DP = 144
R = 5
BLK = 1000
NW = 32
EPT = E // NW
K = 80
NCH = EPT // K
N_ACC = 10240
RPT = N_ACC // 16

_DN = (((1,), (1,)), ((), ()))


def _tables_body(xst, xma, xro, xjb, wl, wr, bl, t0, t1, t2, t3, t4, z):
    ones = jnp.ones((BLK, DP - D), jnp.float32)
    srcs = (xst, xst, xma, xma, xro)
    outs = (t0, t1, t2, t3, t4)
    for r in range(R):
        y = lax.dot_general(srcs[r][...], wl[r], _DN,
                            preferred_element_type=jnp.float32)
        outs[r][...] = jnp.concatenate([y, ones], axis=1)
    wrs = jnp.sum(wr[...], axis=0)
    bls = jnp.sum(bl[...], axis=0)
    z[...] = lax.dot_general(xjb[...], wrs, _DN,
                             preferred_element_type=jnp.float32) + bls[None, :]


def _build_tables(xst, xma, xro, xjb, wl, wr, bl):
    row = pl.BlockSpec((BLK, D), lambda b: (b, 0))
    full3 = pl.BlockSpec((R, D, D), lambda b: (0, 0, 0))
    return pl.pallas_call(
        _tables_body,
        grid=(N // BLK,),
        in_specs=[row, row, row, row, full3, full3,
                  pl.BlockSpec((R, D), lambda b: (0, 0))],
        out_specs=[pl.BlockSpec((BLK, DP), lambda b: (b, 0))] * R
                  + [pl.BlockSpec((BLK, D), lambda b: (b, 0))],
        out_shape=[jax.ShapeDtypeStruct((N, DP), jnp.float32)] * R
                  + [jax.ShapeDtypeStruct((N, D), jnp.float32)],
    )(xst, xma, xro, xjb, wl, wr, bl)


@functools.partial(
    pl.kernel,
    out_type=jax.ShapeDtypeStruct((R, 2, N_ACC, DP), jnp.float32),
    mesh=plsc.VectorSubcoreMesh(core_axis_name="c", subcore_axis_name="s"),
    compiler_params=pltpu.CompilerParams(use_tc_tiling_on_sc=False),
    scratch_types=[
        pltpu.VMEM_SHARED((N_ACC, DP), jnp.float32),
        pltpu.VMEM((K,), jnp.int32),
        pltpu.VMEM((K,), jnp.int32),
        pltpu.VMEM((K, DP), jnp.float32),
        pltpu.SemaphoreType.DMA,
    ],
)
def _sc_segment_sums(t0, t1, t2, t3, t4,
                     s0, s1, s2, s3, s4,
                     d0, d1, d2, d3, d4,
                     zrows, acc_out,
                     acc_sh, src_v, dst_v, rows_v, sem):
    c = lax.axis_index("c")
    s = lax.axis_index("s")
    wid = s * 2 + c
    row0 = s * RPT
    tabs = (t0, t1, t2, t3, t4)
    srcs = (s0, s1, s2, s3, s4)
    dsts = (d0, d1, d2, d3, d4)

    for r in range(R):
        pltpu.sync_copy(zrows, acc_sh.at[pl.ds(row0, RPT)])
        plsc.subcore_barrier()

        def chunk(j, carry, r=r):
            base = pl.multiple_of(wid * EPT + j * K, 8)
            pltpu.sync_copy(srcs[r].at[pl.ds(base, K)], src_v)
            pltpu.sync_copy(dsts[r].at[pl.ds(base, K)], dst_v)
            pltpu.async_copy(tabs[r].at[src_v], rows_v, sem).wait()
            pltpu.sync_copy(rows_v, acc_sh.at[dst_v], add=True)
            return carry

        lax.fori_loop(0, NCH, chunk, 0)
        plsc.subcore_barrier()
        pltpu.sync_copy(acc_sh.at[pl.ds(row0, RPT)],
                        acc_out.at[r, c, pl.ds(row0, RPT)])
        plsc.subcore_barrier()


def _combine_body(acc, z, o):
    out = z[...]
    for r in range(R):
        tot = acc[r, 0] + acc[r, 1]
        cnt = tot[:, D:D + 1]
        out = out + tot[:, :D] / jnp.maximum(cnt, 1.0)
    o[...] = jnp.maximum(out, 0.0)


def _combine(acc, z):
    return pl.pallas_call(
        _combine_body,
        grid=(N // BLK,),
        in_specs=[pl.BlockSpec((R, 2, BLK, DP), lambda b: (0, 0, b, 0)),
                  pl.BlockSpec((BLK, D), lambda b: (b, 0))],
        out_specs=pl.BlockSpec((BLK, D), lambda b: (b, 0)),
        out_shape=jax.ShapeDtypeStruct((N, D), jnp.float32),
    )(acc, z)


def kernel(x_station, x_machine, x_robot, x_job,
           edge_can_load, edge_loaded, edge_will_execute, edge_execute,
           edge_hold,
           Wl_can_load, bl_can_load, Wr_can_load,
           Wl_loaded, bl_loaded, Wr_loaded,
           Wl_will_execute, bl_will_execute, Wr_will_execute,
           Wl_execute, bl_execute, Wr_execute,
           Wl_hold, bl_hold, Wr_hold):
    edges = (edge_can_load, edge_loaded, edge_will_execute, edge_execute,
             edge_hold)
    srcs = [e[0].astype(jnp.int32) for e in edges]
    dsts = [e[1].astype(jnp.int32) for e in edges]
    wl = jnp.stack([Wl_can_load, Wl_loaded, Wl_will_execute, Wl_execute,
                    Wl_hold])
    wr = jnp.stack([Wr_can_load, Wr_loaded, Wr_will_execute, Wr_execute,
                    Wr_hold])
    bl = jnp.stack([bl_can_load, bl_loaded, bl_will_execute, bl_execute,
                    bl_hold])

    t0, t1, t2, t3, t4, z = _build_tables(x_station, x_machine, x_robot,
                                          x_job, wl, wr, bl)
    zrows = jnp.zeros((RPT, DP), jnp.float32)
    acc = _sc_segment_sums(t0, t1, t2, t3, t4, *srcs, *dsts, zrows)
    return _combine(acc, z)


# speedup vs baseline: 6.7943x; 1.6350x over previous
import functools

import jax
import jax.numpy as jnp
from jax import lax
from jax.experimental import pallas as pl
from jax.experimental.pallas import tpu as pltpu
from jax.experimental.pallas import tpu_sc as plsc

N = 10000
E = 320000
D = 128
DP = 144
R = 5
BLK = 1000
NW = 32
EPT = E // NW
K = 50
NCH = EPT // K
RPT = N // 16

_DN = (((1,), (1,)), ((), ()))


def _tables_body(xst, xma, xro, xjb, wl, wr, bl, t0, t1, t2, t3, t4, z):
    ones = jnp.ones((BLK, DP - D), jnp.float32)
    srcs = (xst, xst, xma, xma, xro)
    outs = (t0, t1, t2, t3, t4)
    for r in range(R):
        y = lax.dot_general(srcs[r][...], wl[r], _DN,
                            preferred_element_type=jnp.float32)
        outs[r][...] = jnp.concatenate([y, ones], axis=1)
    wrs = jnp.sum(wr[...], axis=0)
    bls = jnp.sum(bl[...], axis=0)
    z[...] = lax.dot_general(xjb[...], wrs, _DN,
                             preferred_element_type=jnp.float32) + bls[None, :]


def _build_tables(xst, xma, xro, xjb, wl, wr, bl):
    row = pl.BlockSpec((BLK, D), lambda b: (b, 0))
    full3 = pl.BlockSpec((R, D, D), lambda b: (0, 0, 0))
    return pl.pallas_call(
        _tables_body,
        grid=(N // BLK,),
        in_specs=[row, row, row, row, full3, full3,
                  pl.BlockSpec((R, D), lambda b: (0, 0))],
        out_specs=[pl.BlockSpec((BLK, DP), lambda b: (b, 0))] * R
                  + [pl.BlockSpec((BLK, D), lambda b: (b, 0))],
        out_shape=[jax.ShapeDtypeStruct((N, DP), jnp.float32)] * R
                  + [jax.ShapeDtypeStruct((N, D), jnp.float32)],
    )(xst, xma, xro, xjb, wl, wr, bl)


@functools.partial(
    pl.kernel,
    out_type=jax.ShapeDtypeStruct((R, 2, N, DP), jnp.float32),
    mesh=plsc.VectorSubcoreMesh(core_axis_name="c", subcore_axis_name="s"),
    compiler_params=pltpu.CompilerParams(use_tc_tiling_on_sc=False),
    scratch_types=[
        pltpu.VMEM_SHARED((N, DP), jnp.float32),
        pltpu.VMEM((NCH, K), jnp.int32),
        pltpu.VMEM((NCH, K), jnp.int32),
        pltpu.VMEM((K, DP), jnp.float32),
        pltpu.VMEM((K, DP), jnp.float32),
        pltpu.SemaphoreType.DMA,
        pltpu.SemaphoreType.DMA,
        pltpu.SemaphoreType.DMA,
        pltpu.SemaphoreType.DMA,
    ],
)
def _sc_segment_sums(t0, t1, t2, t3, t4,
                     s0, s1, s2, s3, s4,
                     d0, d1, d2, d3, d4,
                     zrows, acc_out,
                     acc_sh, srcbuf, dstbuf, rows0, rows1, g0, g1, w0, w1):
    c = lax.axis_index("c")
    s = lax.axis_index("s")
    wid = s * 2 + c
    row0 = s * RPT
    tabs = (t0, t1, t2, t3, t4)
    srcs = (s0, s1, s2, s3, s4)
    dsts = (d0, d1, d2, d3, d4)
    rows = (rows0, rows1)
    gsem = (g0, g1)
    ssem = (w0, w1)

    for r in range(R):
        tab = tabs[r]
        pltpu.sync_copy(zrows, acc_sh.at[pl.ds(row0, RPT)])
        pltpu.sync_copy(srcs[r].at[pl.ds(wid * NCH, NCH)], srcbuf)
        pltpu.sync_copy(dsts[r].at[pl.ds(wid * NCH, NCH)], dstbuf)
        plsc.subcore_barrier()

        def gather(ch, b):
            pltpu.async_copy(tab.at[srcbuf.at[ch]], rows[b], gsem[b])

        def gather_wait(b):
            pltpu.make_async_copy(tab.at[srcbuf.at[0]], rows[b],
                                  gsem[b]).wait()

        def scatter(ch, b):
            pltpu.async_copy(rows[b], acc_sh.at[dstbuf.at[ch]], ssem[b],
                             add=True)

        def scatter_wait(ch, b):
            pltpu.make_async_copy(rows[b], acc_sh.at[dstbuf.at[ch]],
                                  ssem[b]).wait()

        gather(0, 0)
        gather(1, 1)

        def pair(p, carry):
            for b in range(2):
                ch = 2 * p + b
                gather_wait(b)
                scatter(ch, b)
                scatter_wait(ch, b)
                gather(ch + 2, b)
            return carry

        lax.fori_loop(0, NCH // 2 - 1, pair, 0)
        for b in range(2):
            ch = NCH - 2 + b
            gather_wait(b)
            scatter(ch, b)
            scatter_wait(ch, b)

        plsc.subcore_barrier()
        pltpu.sync_copy(acc_sh.at[pl.ds(row0, RPT)],
                        acc_out.at[r, c, pl.ds(row0, RPT)])
    plsc.subcore_barrier()


def _combine_body(acc, z, o):
    out = z[...]
    for r in range(R):
        tot = acc[r, 0] + acc[r, 1]
        cnt = tot[:, D:D + 1]
        out = out + tot[:, :D] / jnp.maximum(cnt, 1.0)
    o[...] = jnp.maximum(out, 0.0)


def _combine(acc, z):
    return pl.pallas_call(
        _combine_body,
        grid=(N // BLK,),
        in_specs=[pl.BlockSpec((R, 2, BLK, DP), lambda b: (0, 0, b, 0)),
                  pl.BlockSpec((BLK, D), lambda b: (b, 0))],
        out_specs=pl.BlockSpec((BLK, D), lambda b: (b, 0)),
        out_shape=jax.ShapeDtypeStruct((N, D), jnp.float32),
    )(acc, z)


def kernel(x_station, x_machine, x_robot, x_job,
           edge_can_load, edge_loaded, edge_will_execute, edge_execute,
           edge_hold,
           Wl_can_load, bl_can_load, Wr_can_load,
           Wl_loaded, bl_loaded, Wr_loaded,
           Wl_will_execute, bl_will_execute, Wr_will_execute,
           Wl_execute, bl_execute, Wr_execute,
           Wl_hold, bl_hold, Wr_hold):
    edges = (edge_can_load, edge_loaded, edge_will_execute, edge_execute,
             edge_hold)
    srcs = [e[0].astype(jnp.int32).reshape(E // K, K) for e in edges]
    dsts = [e[1].astype(jnp.int32).reshape(E // K, K) for e in edges]
    wl = jnp.stack([Wl_can_load, Wl_loaded, Wl_will_execute, Wl_execute,
                    Wl_hold])
    wr = jnp.stack([Wr_can_load, Wr_loaded, Wr_will_execute, Wr_execute,
                    Wr_hold])
    bl = jnp.stack([bl_can_load, bl_loaded, bl_will_execute, bl_execute,
                    bl_hold])

    t0, t1, t2, t3, t4, z = _build_tables(x_station, x_machine, x_robot,
                                          x_job, wl, wr, bl)
    zrows = jnp.zeros((RPT, DP), jnp.float32)
    acc = _sc_segment_sums(t0, t1, t2, t3, t4, *srcs, *dsts, zrows)
    return _combine(acc, z)


# speedup vs baseline: 7.4557x; 1.0973x over previous
import functools

import jax
import jax.numpy as jnp
from jax import lax
from jax.experimental import pallas as pl
from jax.experimental.pallas import tpu as pltpu
from jax.experimental.pallas import tpu_sc as plsc

N = 10000
E = 320000
D = 128
DP = 144
R = 5
BLK = 1000
NW = 32
EPT = E // NW
K = 40
NCH = EPT // K
RPT = N // 16

_DN = (((1,), (1,)), ((), ()))


def _tables_body(xst, xma, xro, xjb,
                 wl0, wl1, wl2, wl3, wl4,
                 wr0, wr1, wr2, wr3, wr4,
                 bl0, bl1, bl2, bl3, bl4,
                 t0, t1, t2, t3, t4, z):
    ones = jnp.ones((BLK, DP - D), jnp.float32)
    srcs = (xst, xst, xma, xma, xro)
    wls = (wl0, wl1, wl2, wl3, wl4)
    outs = (t0, t1, t2, t3, t4)
    for r in range(R):
        y = lax.dot_general(srcs[r][...], wls[r][...], _DN,
                            preferred_element_type=jnp.float32)
        outs[r][...] = jnp.concatenate([y, ones], axis=1)
    wrs = wr0[...] + wr1[...] + wr2[...] + wr3[...] + wr4[...]
    bls = bl0[...] + bl1[...] + bl2[...] + bl3[...] + bl4[...]
    z[...] = lax.dot_general(xjb[...], wrs, _DN,
                             preferred_element_type=jnp.float32) + bls[None, :]


def _build_tables(xst, xma, xro, xjb, wls, wrs, bls):
    row = pl.BlockSpec((BLK, D), lambda b: (b, 0))
    mat = pl.BlockSpec((D, D), lambda b: (0, 0))
    vec = pl.BlockSpec((D,), lambda b: (0,))
    return pl.pallas_call(
        _tables_body,
        grid=(N // BLK,),
        in_specs=[row, row, row, row] + [mat] * R + [mat] * R + [vec] * R,
        out_specs=[pl.BlockSpec((BLK, DP), lambda b: (b, 0))] * R
                  + [pl.BlockSpec((BLK, D), lambda b: (b, 0))],
        out_shape=[jax.ShapeDtypeStruct((N, DP), jnp.float32)] * R
                  + [jax.ShapeDtypeStruct((N, D), jnp.float32)],
    )(xst, xma, xro, xjb, *wls, *wrs, *bls)


@functools.partial(
    pl.kernel,
    out_type=jax.ShapeDtypeStruct((R, 2, N, DP), jnp.float32),
    mesh=plsc.VectorSubcoreMesh(core_axis_name="c", subcore_axis_name="s"),
    compiler_params=pltpu.CompilerParams(use_tc_tiling_on_sc=False),
    scratch_types=[
        pltpu.VMEM_SHARED((N, DP), jnp.float32),
        pltpu.VMEM((EPT,), jnp.int32),
        pltpu.VMEM((EPT,), jnp.int32),
        pltpu.VMEM((K, DP), jnp.float32),
        pltpu.VMEM((K, DP), jnp.float32),
        pltpu.SemaphoreType.DMA,
        pltpu.SemaphoreType.DMA,
        pltpu.SemaphoreType.DMA,
        pltpu.SemaphoreType.DMA,
    ],
)
def _sc_segment_sums(e0, e1, e2, e3, e4,
                     t0, t1, t2, t3, t4,
                     zrows, acc_out,
                     acc_sh, srcbuf, dstbuf, rows0, rows1, g0, g1, w0, w1):
    c = lax.axis_index("c")
    s = lax.axis_index("s")
    wid = s * 2 + c
    row0 = s * RPT
    tabs = (t0, t1, t2, t3, t4)
    eds = (e0, e1, e2, e3, e4)
    rows = (rows0, rows1)
    gsem = (g0, g1)
    ssem = (w0, w1)

    for r in range(R):
        tab = tabs[r]
        pltpu.sync_copy(zrows, acc_sh.at[pl.ds(row0, RPT)])
        pltpu.sync_copy(eds[r].at[0, pl.ds(wid * EPT, EPT)], srcbuf)
        pltpu.sync_copy(eds[r].at[1, pl.ds(wid * EPT, EPT)], dstbuf)
        plsc.subcore_barrier()

        def gather(ch, b):
            pltpu.async_copy(tab.at[srcbuf.at[pl.ds(ch * K, K)]], rows[b],
                             gsem[b])

        def gather_wait(b):
            pltpu.make_async_copy(tab.at[srcbuf.at[pl.ds(0, K)]], rows[b],
                                  gsem[b]).wait()

        def scatter(ch, b):
            pltpu.async_copy(rows[b], acc_sh.at[dstbuf.at[pl.ds(ch * K, K)]],
                             ssem[b], add=True)

        def scatter_wait(ch, b):
            pltpu.make_async_copy(rows[b],
                                  acc_sh.at[dstbuf.at[pl.ds(ch * K, K)]],
                                  ssem[b]).wait()

        gather(0, 0)
        gather(1, 1)

        def pair(p, carry):
            for b in range(2):
                ch = 2 * p + b
                gather_wait(b)
                scatter(ch, b)
                scatter_wait(ch, b)
                gather(ch + 2, b)
            return carry

        lax.fori_loop(0, NCH // 2 - 1, pair, 0)
        for b in range(2):
            ch = NCH - 2 + b
            gather_wait(b)
            scatter(ch, b)
            scatter_wait(ch, b)

        plsc.subcore_barrier()
        pltpu.sync_copy(acc_sh.at[pl.ds(row0, RPT)],
                        acc_out.at[r, c, pl.ds(row0, RPT)])
    plsc.subcore_barrier()


def _combine_body(acc, z, o):
    out = z[...]
    for r in range(R):
        tot = acc[r, 0] + acc[r, 1]
        cnt = tot[:, D:D + 1]
        out = out + tot[:, :D] / jnp.maximum(cnt, 1.0)
    o[...] = jnp.maximum(out, 0.0)


def _combine(acc, z):
    return pl.pallas_call(
        _combine_body,
        grid=(N // BLK,),
        in_specs=[pl.BlockSpec((R, 2, BLK, DP), lambda b: (0, 0, b, 0)),
                  pl.BlockSpec((BLK, D), lambda b: (b, 0))],
        out_specs=pl.BlockSpec((BLK, D), lambda b: (b, 0)),
        out_shape=jax.ShapeDtypeStruct((N, D), jnp.float32),
    )(acc, z)


def kernel(x_station, x_machine, x_robot, x_job,
           edge_can_load, edge_loaded, edge_will_execute, edge_execute,
           edge_hold,
           Wl_can_load, bl_can_load, Wr_can_load,
           Wl_loaded, bl_loaded, Wr_loaded,
           Wl_will_execute, bl_will_execute, Wr_will_execute,
           Wl_execute, bl_execute, Wr_execute,
           Wl_hold, bl_hold, Wr_hold):
    edges = (edge_can_load, edge_loaded, edge_will_execute, edge_execute,
             edge_hold)
    ei = [e.astype(jnp.int32) for e in edges]
    wls = (Wl_can_load, Wl_loaded, Wl_will_execute, Wl_execute, Wl_hold)
    wrs = (Wr_can_load, Wr_loaded, Wr_will_execute, Wr_execute, Wr_hold)
    bls = (bl_can_load, bl_loaded, bl_will_execute, bl_execute, bl_hold)

    t0, t1, t2, t3, t4, z = _build_tables(x_station, x_machine, x_robot,
                                          x_job, wls, wrs, bls)
    zrows = jnp.zeros((RPT, DP), jnp.float32)
    acc = _sc_segment_sums(*ei, t0, t1, t2, t3, t4, zrows)
    return _combine(acc, z)


# speedup vs baseline: 8.6374x; 1.1585x over previous
import functools

import jax
import jax.numpy as jnp
from jax import lax
from jax.experimental import pallas as pl
from jax.experimental.pallas import tpu as pltpu
from jax.experimental.pallas import tpu_sc as plsc

N = 10000
E = 320000
D = 128
DP = 144
R = 5
BLK = 1000
NW = 32
EPT = E // NW
K = 40
NCH = EPT // K
RPT = N // 16

_DN = (((1,), (1,)), ((), ()))


def _tables_body(xst, xma, xro, xjb,
                 wl0, wl1, wl2, wl3, wl4,
                 wr0, wr1, wr2, wr3, wr4,
                 bl0, bl1, bl2, bl3, bl4,
                 t0, t1, t2, t3, t4, z):
    ones = jnp.ones((BLK, DP - D), jnp.float32)
    srcs = (xst, xst, xma, xma, xro)
    wls = (wl0, wl1, wl2, wl3, wl4)
    outs = (t0, t1, t2, t3, t4)
    for r in range(R):
        y = lax.dot_general(srcs[r][...], wls[r][...], _DN,
                            preferred_element_type=jnp.float32)
        outs[r][...] = jnp.concatenate([y, ones], axis=1)
    wrs = wr0[...] + wr1[...] + wr2[...] + wr3[...] + wr4[...]
    bls = bl0[...] + bl1[...] + bl2[...] + bl3[...] + bl4[...]
    z[...] = lax.dot_general(xjb[...], wrs, _DN,
                             preferred_element_type=jnp.float32) + bls[None, :]


def _build_tables(xst, xma, xro, xjb, wls, wrs, bls):
    row = pl.BlockSpec((BLK, D), lambda b: (b, 0))
    mat = pl.BlockSpec((D, D), lambda b: (0, 0))
    vec = pl.BlockSpec((D,), lambda b: (0,))
    return pl.pallas_call(
        _tables_body,
        grid=(N // BLK,),
        in_specs=[row, row, row, row] + [mat] * R + [mat] * R + [vec] * R,
        out_specs=[pl.BlockSpec((BLK, DP), lambda b: (b, 0))] * R
                  + [pl.BlockSpec((BLK, D), lambda b: (b, 0))],
        out_shape=[jax.ShapeDtypeStruct((N, DP), jnp.float32)] * R
                  + [jax.ShapeDtypeStruct((N, D), jnp.float32)],
    )(xst, xma, xro, xjb, *wls, *wrs, *bls)


@functools.partial(
    pl.kernel,
    out_type=jax.ShapeDtypeStruct((R, 2, N, DP), jnp.float32),
    mesh=plsc.VectorSubcoreMesh(core_axis_name="c", subcore_axis_name="s"),
    compiler_params=pltpu.CompilerParams(use_tc_tiling_on_sc=False),
    scratch_types=[
        pltpu.VMEM_SHARED((N, DP), jnp.float32),
        pltpu.VMEM((EPT,), jnp.int32),
        pltpu.VMEM((EPT,), jnp.int32),
        pltpu.VMEM((K, DP), jnp.float32),
        pltpu.VMEM((K, DP), jnp.float32),
        pltpu.VMEM((K, DP), jnp.float32),
        pltpu.SemaphoreType.DMA,
        pltpu.SemaphoreType.DMA,
        pltpu.SemaphoreType.DMA,
        pltpu.SemaphoreType.DMA,
        pltpu.SemaphoreType.DMA,
        pltpu.SemaphoreType.DMA,
    ],
)
def _sc_segment_sums(e0, e1, e2, e3, e4,
                     t0, t1, t2, t3, t4,
                     zrows, acc_out,
                     acc_sh, srcbuf, dstbuf, rows0, rows1, rows2,
                     g0, g1, g2, w0, w1, w2):
    c = lax.axis_index("c")
    s = lax.axis_index("s")
    wid = s * 2 + c
    row0 = s * RPT
    tabs = (t0, t1, t2, t3, t4)
    eds = (e0, e1, e2, e3, e4)
    rows = (rows0, rows1, rows2)
    gsem = (g0, g1, g2)
    ssem = (w0, w1, w2)

    for r in range(R):
        tab = tabs[r]
        pltpu.sync_copy(zrows, acc_sh.at[pl.ds(row0, RPT)])
        pltpu.sync_copy(eds[r].at[0, pl.ds(wid * EPT, EPT)], srcbuf)
        pltpu.sync_copy(eds[r].at[1, pl.ds(wid * EPT, EPT)], dstbuf)
        plsc.subcore_barrier()

        def gather(ch, b):
            pltpu.async_copy(tab.at[srcbuf.at[pl.ds(ch * K, K)]], rows[b],
                             gsem[b])

        def gather_wait(b):
            pltpu.make_async_copy(tab.at[srcbuf.at[pl.ds(0, K)]], rows[b],
                                  gsem[b]).wait()

        def scatter(ch, b):
            pltpu.async_copy(rows[b], acc_sh.at[dstbuf.at[pl.ds(ch * K, K)]],
                             ssem[b], add=True)

        def scatter_wait(ch, b):
            pltpu.make_async_copy(rows[b],
                                  acc_sh.at[dstbuf.at[pl.ds(ch * K, K)]],
                                  ssem[b]).wait()

        gather(0, 0)
        gather(1, 1)
        gather_wait(0)
        scatter(0, 0)
        gather(2, 2)
        gather_wait(1)
        scatter(1, 1)
        scatter_wait(0, 0)
        gather(3, 0)

        def six(p, carry):
            for u in range(6):
                ch = 2 + 6 * p + u
                b = (2 + u) % 3
                gather_wait(b)
                scatter(ch, b)
                scatter_wait(ch - 1, (b + 2) % 3)
                gather(ch + 2, (b + 2) % 3)
            return carry

        lax.fori_loop(0, (NCH - 4) // 6, six, 0)
        for ch in (NCH - 2, NCH - 1):
            b = ch % 3
            gather_wait(b)
            scatter(ch, b)
            scatter_wait(ch - 1, (b + 2) % 3)
        scatter_wait(NCH - 1, (NCH - 1) % 3)

        plsc.subcore_barrier()
        pltpu.sync_copy(acc_sh.at[pl.ds(row0, RPT)],
                        acc_out.at[r, c, pl.ds(row0, RPT)])
    plsc.subcore_barrier()


def _combine_body(acc, z, o):
    out = z[...]
    for r in range(R):
        tot = acc[r, 0] + acc[r, 1]
        cnt = tot[:, D:D + 1]
        out = out + tot[:, :D] / jnp.maximum(cnt, 1.0)
    o[...] = jnp.maximum(out, 0.0)


def _combine(acc, z):
    return pl.pallas_call(
        _combine_body,
        grid=(N // BLK,),
        in_specs=[pl.BlockSpec((R, 2, BLK, DP), lambda b: (0, 0, b, 0)),
                  pl.BlockSpec((BLK, D), lambda b: (b, 0))],
        out_specs=pl.BlockSpec((BLK, D), lambda b: (b, 0)),
        out_shape=jax.ShapeDtypeStruct((N, D), jnp.float32),
    )(acc, z)


def kernel(x_station, x_machine, x_robot, x_job,
           edge_can_load, edge_loaded, edge_will_execute, edge_execute,
           edge_hold,
           Wl_can_load, bl_can_load, Wr_can_load,
           Wl_loaded, bl_loaded, Wr_loaded,
           Wl_will_execute, bl_will_execute, Wr_will_execute,
           Wl_execute, bl_execute, Wr_execute,
           Wl_hold, bl_hold, Wr_hold):
    edges = (edge_can_load, edge_loaded, edge_will_execute, edge_execute,
             edge_hold)
    ei = [e.astype(jnp.int32) for e in edges]
    wls = (Wl_can_load, Wl_loaded, Wl_will_execute, Wl_execute, Wl_hold)
    wrs = (Wr_can_load, Wr_loaded, Wr_will_execute, Wr_execute, Wr_hold)
    bls = (bl_can_load, bl_loaded, bl_will_execute, bl_execute, bl_hold)

    t0, t1, t2, t3, t4, z = _build_tables(x_station, x_machine, x_robot,
                                          x_job, wls, wrs, bls)
    zrows = jnp.zeros((RPT, DP), jnp.float32)
    acc = _sc_segment_sums(*ei, t0, t1, t2, t3, t4, zrows)
    return _combine(acc, z)


# speedup vs baseline: 9.9858x; 1.1561x over previous
import functools

import jax
import jax.numpy as jnp
from jax import lax
from jax.experimental import pallas as pl
from jax.experimental.pallas import tpu as pltpu
from jax.experimental.pallas import tpu_sc as plsc

N = 10000
E = 320000
D = 128
CW = 16
R = 5
BLK = 1000
NW = 32
EPT = E // NW
K = 40
NCH = EPT // K
RPT = N // 16

_DN = (((1,), (1,)), ((), ()))


def _tables_body(xst, xma, xro, xjb,
                 wl0, wl1, wl2, wl3, wl4,
                 wr0, wr1, wr2, wr3, wr4,
                 bl0, bl1, bl2, bl3, bl4,
                 t0, t1, t2, t3, t4, z):
    srcs = (xst, xst, xma, xma, xro)
    wls = (wl0, wl1, wl2, wl3, wl4)
    outs = (t0, t1, t2, t3, t4)
    for r in range(R):
        outs[r][...] = lax.dot_general(srcs[r][...], wls[r][...], _DN,
                                       preferred_element_type=jnp.float32)
    wrs = wr0[...] + wr1[...] + wr2[...] + wr3[...] + wr4[...]
    bls = bl0[...] + bl1[...] + bl2[...] + bl3[...] + bl4[...]
    z[...] = lax.dot_general(xjb[...], wrs, _DN,
                             preferred_element_type=jnp.float32) + bls[None, :]


def _build_tables(xst, xma, xro, xjb, wls, wrs, bls):
    row = pl.BlockSpec((BLK, D), lambda b: (b, 0))
    mat = pl.BlockSpec((D, D), lambda b: (0, 0))
    vec = pl.BlockSpec((D,), lambda b: (0,))
    return pl.pallas_call(
        _tables_body,
        grid=(N // BLK,),
        in_specs=[row, row, row, row] + [mat] * R + [mat] * R + [vec] * R,
        out_specs=[pl.BlockSpec((BLK, D), lambda b: (b, 0))] * (R + 1),
        out_shape=[jax.ShapeDtypeStruct((N, D), jnp.float32)] * (R + 1),
    )(xst, xma, xro, xjb, *wls, *wrs, *bls)


@functools.partial(
    pl.kernel,
    out_type=[jax.ShapeDtypeStruct((R, 2, N, D), jnp.float32),
              jax.ShapeDtypeStruct((R, 2, N, CW), jnp.float32)],
    mesh=plsc.VectorSubcoreMesh(core_axis_name="c", subcore_axis_name="s"),
    compiler_params=pltpu.CompilerParams(use_tc_tiling_on_sc=False),
    scratch_types=[
        pltpu.VMEM_SHARED((N, D), jnp.float32),
        pltpu.VMEM_SHARED((N, CW), jnp.float32),
        pltpu.VMEM((EPT,), jnp.int32),
        pltpu.VMEM((EPT,), jnp.int32),
        pltpu.VMEM((K, CW), jnp.float32),
        pltpu.VMEM((K, D), jnp.float32),
        pltpu.VMEM((K, D), jnp.float32),
        pltpu.VMEM((K, D), jnp.float32),
        pltpu.SemaphoreType.DMA,
        pltpu.SemaphoreType.DMA,
        pltpu.SemaphoreType.DMA,
        pltpu.SemaphoreType.DMA,
        pltpu.SemaphoreType.DMA,
        pltpu.SemaphoreType.DMA,
        pltpu.SemaphoreType.DMA,
        pltpu.SemaphoreType.DMA,
        pltpu.SemaphoreType.DMA,
    ],
)
def _sc_segment_sums(e0, e1, e2, e3, e4,
                     t0, t1, t2, t3, t4,
                     zrows, zcnt, ones_h, acc_out, cnt_out,
                     acc_sh, cnt_sh, srcbuf, dstbuf, onesbuf,
                     rows0, rows1, rows2,
                     g0, g1, g2, w0, w1, w2, c0, c1, c2):
    c = lax.axis_index("c")
    s = lax.axis_index("s")
    wid = s * 2 + c
    row0 = s * RPT
    tabs = (t0, t1, t2, t3, t4)
    eds = (e0, e1, e2, e3, e4)
    rows = (rows0, rows1, rows2)
    gsem = (g0, g1, g2)
    ssem = (w0, w1, w2)
    csem = (c0, c1, c2)

    pltpu.sync_copy(ones_h, onesbuf)

    for r in range(R):
        tab = tabs[r]
        pltpu.sync_copy(zrows, acc_sh.at[pl.ds(row0, RPT)])
        pltpu.sync_copy(zcnt, cnt_sh.at[pl.ds(row0, RPT)])
        pltpu.sync_copy(eds[r].at[0, pl.ds(wid * EPT, EPT)], srcbuf)
        pltpu.sync_copy(eds[r].at[1, pl.ds(wid * EPT, EPT)], dstbuf)
        plsc.subcore_barrier()

        def gather(ch, b):
            pltpu.async_copy(tab.at[srcbuf.at[pl.ds(ch * K, K)]], rows[b],
                             gsem[b])

        def gather_wait(b):
            pltpu.make_async_copy(tab.at[srcbuf.at[pl.ds(0, K)]], rows[b],
                                  gsem[b]).wait()

        def scatter(ch, b):
            idx = dstbuf.at[pl.ds(ch * K, K)]
            pltpu.async_copy(rows[b], acc_sh.at[idx], ssem[b], add=True)
            pltpu.async_copy(onesbuf, cnt_sh.at[idx], csem[b], add=True)

        def scatter_wait(ch, b):
            idx = dstbuf.at[pl.ds(ch * K, K)]
            pltpu.make_async_copy(rows[b], acc_sh.at[idx], ssem[b]).wait()
            pltpu.make_async_copy(onesbuf, cnt_sh.at[idx], csem[b]).wait()

        gather(0, 0)
        gather(1, 1)
        gather_wait(0)
        scatter(0, 0)
        gather(2, 2)
        gather_wait(1)
        scatter(1, 1)
        scatter_wait(0, 0)
        gather(3, 0)

        def six(p, carry):
            for u in range(6):
                ch = 2 + 6 * p + u
                b = (2 + u) % 3
                gather_wait(b)
                scatter(ch, b)
                scatter_wait(ch - 1, (b + 2) % 3)
                gather(ch + 2, (b + 2) % 3)
            return carry

        lax.fori_loop(0, (NCH - 4) // 6, six, 0)
        for ch in (NCH - 2, NCH - 1):
            b = ch % 3
            gather_wait(b)
            scatter(ch, b)
            scatter_wait(ch - 1, (b + 2) % 3)
        scatter_wait(NCH - 1, (NCH - 1) % 3)

        plsc.subcore_barrier()
        pltpu.sync_copy(acc_sh.at[pl.ds(row0, RPT)],
                        acc_out.at[r, c, pl.ds(row0, RPT)])
        pltpu.sync_copy(cnt_sh.at[pl.ds(row0, RPT)],
                        cnt_out.at[r, c, pl.ds(row0, RPT)])
    plsc.subcore_barrier()


def _combine_body(acc, cnt, z, o):
    out = z[...]
    for r in range(R):
        tot = acc[r, 0] + acc[r, 1]
        cr = cnt[r, 0, :, 0:1] + cnt[r, 1, :, 0:1]
        out = out + tot / jnp.maximum(cr, 1.0)
    o[...] = jnp.maximum(out, 0.0)


def _combine(acc, cnt, z):
    return pl.pallas_call(
        _combine_body,
        grid=(N // BLK,),
        in_specs=[pl.BlockSpec((R, 2, BLK, D), lambda b: (0, 0, b, 0)),
                  pl.BlockSpec((R, 2, BLK, CW), lambda b: (0, 0, b, 0)),
                  pl.BlockSpec((BLK, D), lambda b: (b, 0))],
        out_specs=pl.BlockSpec((BLK, D), lambda b: (b, 0)),
        out_shape=jax.ShapeDtypeStruct((N, D), jnp.float32),
    )(acc, cnt, z)


def kernel(x_station, x_machine, x_robot, x_job,
           edge_can_load, edge_loaded, edge_will_execute, edge_execute,
           edge_hold,
           Wl_can_load, bl_can_load, Wr_can_load,
           Wl_loaded, bl_loaded, Wr_loaded,
           Wl_will_execute, bl_will_execute, Wr_will_execute,
           Wl_execute, bl_execute, Wr_execute,
           Wl_hold, bl_hold, Wr_hold):
    edges = (edge_can_load, edge_loaded, edge_will_execute, edge_execute,
             edge_hold)
    ei = [e.astype(jnp.int32) for e in edges]
    wls = (Wl_can_load, Wl_loaded, Wl_will_execute, Wl_execute, Wl_hold)
    wrs = (Wr_can_load, Wr_loaded, Wr_will_execute, Wr_execute, Wr_hold)
    bls = (bl_can_load, bl_loaded, bl_will_execute, bl_execute, bl_hold)

    t0, t1, t2, t3, t4, z = _build_tables(x_station, x_machine, x_robot,
                                          x_job, wls, wrs, bls)
    zrows = jnp.zeros((RPT, D), jnp.float32)
    zcnt = jnp.zeros((RPT, CW), jnp.float32)
    ones_h = jnp.ones((K, CW), jnp.float32)
    acc, cnt = _sc_segment_sums(*ei, t0, t1, t2, t3, t4, zrows, zcnt, ones_h)
    return _combine(acc, cnt, z)


# speedup vs baseline: 12.4605x; 1.2478x over previous
import functools

import jax
import jax.numpy as jnp
from jax import lax
from jax.experimental import pallas as pl
from jax.experimental.pallas import tpu as pltpu
from jax.experimental.pallas import tpu_sc as plsc

N = 10000
E = 320000
D = 128
CW = 8
R = 5
BLK = 1000
NW = 32
EPT = E // NW
K = 80
NCH = EPT // K
RPT = N // 16

_DN = (((1,), (1,)), ((), ()))


def _tables_body(xst, xma, xro, xjb,
                 wl0, wl1, wl2, wl3, wl4,
                 wr0, wr1, wr2, wr3, wr4,
                 bl0, bl1, bl2, bl3, bl4,
                 t0, t1, t2, t3, t4, z):
    srcs = (xst, xst, xma, xma, xro)
    wls = (wl0, wl1, wl2, wl3, wl4)
    outs = (t0, t1, t2, t3, t4)
    for r in range(R):
        outs[r][...] = lax.dot_general(srcs[r][...], wls[r][...], _DN,
                                       preferred_element_type=jnp.float32)
    wrs = wr0[...] + wr1[...] + wr2[...] + wr3[...] + wr4[...]
    bls = bl0[...] + bl1[...] + bl2[...] + bl3[...] + bl4[...]
    z[...] = lax.dot_general(xjb[...], wrs, _DN,
                             preferred_element_type=jnp.float32) + bls[None, :]


def _build_tables(xst, xma, xro, xjb, wls, wrs, bls):
    row = pl.BlockSpec((BLK, D), lambda b: (b, 0))
    mat = pl.BlockSpec((D, D), lambda b: (0, 0))
    vec = pl.BlockSpec((D,), lambda b: (0,))
    return pl.pallas_call(
        _tables_body,
        grid=(N // BLK,),
        in_specs=[row, row, row, row] + [mat] * R + [mat] * R + [vec] * R,
        out_specs=[pl.BlockSpec((BLK, D), lambda b: (b, 0))] * (R + 1),
        out_shape=[jax.ShapeDtypeStruct((N, D), jnp.float32)] * (R + 1),
    )(xst, xma, xro, xjb, *wls, *wrs, *bls)


@functools.partial(
    pl.kernel,
    out_type=[jax.ShapeDtypeStruct((R, 2, N, D), jnp.float32),
              jax.ShapeDtypeStruct((R, 2, N, CW), jnp.float32)],
    mesh=plsc.VectorSubcoreMesh(core_axis_name="c", subcore_axis_name="s"),
    compiler_params=pltpu.CompilerParams(use_tc_tiling_on_sc=False),
    scratch_types=[
        pltpu.VMEM_SHARED((N, D), jnp.float32),
        pltpu.VMEM_SHARED((N, CW), jnp.float32),
        pltpu.VMEM((K,), jnp.int32),
        pltpu.VMEM((K,), jnp.int32),
        pltpu.VMEM((K,), jnp.int32),
        pltpu.VMEM((EPT,), jnp.int32),
        pltpu.VMEM((K, CW), jnp.float32),
        pltpu.VMEM((K, D), jnp.float32),
        pltpu.VMEM((K, D), jnp.float32),
        pltpu.VMEM((K, D), jnp.float32),
        pltpu.SemaphoreType.DMA,
        pltpu.SemaphoreType.DMA,
        pltpu.SemaphoreType.DMA,
        pltpu.SemaphoreType.DMA,
        pltpu.SemaphoreType.DMA,
        pltpu.SemaphoreType.DMA,
        pltpu.SemaphoreType.DMA,
        pltpu.SemaphoreType.DMA,
        pltpu.SemaphoreType.DMA,
        pltpu.SemaphoreType.DMA,
        pltpu.SemaphoreType.DMA,
        pltpu.SemaphoreType.DMA,
    ],
)
def _sc_segment_sums(e0, e1, e2, e3, e4,
                     t0, t1, t2, t3, t4,
                     zrows, zcnt, ones_h, acc_out, cnt_out,
                     acc_sh, cnt_sh, sb0, sb1, sb2, dstbuf, onesbuf,
                     rows0, rows1, rows2,
                     g0, g1, g2, w0, w1, w2, c0, c1, c2, i0, i1, i2):
    c = lax.axis_index("c")
    s = lax.axis_index("s")
    wid = s * 2 + c
    row0 = s * RPT
    tabs = (t0, t1, t2, t3, t4)
    eds = (e0, e1, e2, e3, e4)
    srcb = (sb0, sb1, sb2)
    rows = (rows0, rows1, rows2)
    gsem = (g0, g1, g2)
    ssem = (w0, w1, w2)
    csem = (c0, c1, c2)
    isem = (i0, i1, i2)

    pltpu.sync_copy(ones_h, onesbuf)

    for r in range(R):
        tab = tabs[r]
        ed = eds[r]
        ebase = wid * EPT
        pltpu.sync_copy(zrows, acc_sh.at[pl.ds(row0, RPT)])
        pltpu.sync_copy(zcnt, cnt_sh.at[pl.ds(row0, RPT)])
        pltpu.sync_copy(ed.at[1, pl.ds(ebase, EPT)], dstbuf)
        plsc.subcore_barrier()

        def src_load(ch, b):
            pltpu.async_copy(ed.at[0, pl.ds(ebase + ch * K, K)], srcb[b],
                             isem[b])

        def src_wait(b):
            pltpu.make_async_copy(ed.at[0, pl.ds(ebase, K)], srcb[b],
                                  isem[b]).wait()

        def gather(b):
            pltpu.async_copy(tab.at[srcb[b]], rows[b], gsem[b])

        def gather_wait(b):
            pltpu.make_async_copy(tab.at[srcb[b]], rows[b], gsem[b]).wait()

        def scatter(ch, b):
            idx = dstbuf.at[pl.ds(ch * K, K)]
            pltpu.async_copy(rows[b], acc_sh.at[idx], ssem[b], add=True)
            pltpu.async_copy(onesbuf, cnt_sh.at[idx], csem[b], add=True)

        def scatter_wait(ch, b):
            idx = dstbuf.at[pl.ds(ch * K, K)]
            pltpu.make_async_copy(rows[b], acc_sh.at[idx], ssem[b]).wait()
            pltpu.make_async_copy(onesbuf, cnt_sh.at[idx], csem[b]).wait()

        pltpu.sync_copy(ed.at[0, pl.ds(ebase, K)], sb0)
        pltpu.sync_copy(ed.at[0, pl.ds(ebase + K, K)], sb1)
        src_load(2, 2)
        gather(0)
        gather(1)
        gather_wait(0)
        scatter(0, 0)
        src_wait(2)
        gather(2)
        src_load(3, 0)
        gather_wait(1)
        scatter(1, 1)
        scatter_wait(0, 0)
        src_wait(0)
        gather(0)
        src_load(4, 1)

        def six(p, carry):
            for u in range(6):
                ch = 2 + 6 * p + u
                b = (2 + u) % 3
                b2 = (b + 2) % 3
                gather_wait(b)
                scatter(ch, b)
                scatter_wait(ch - 1, b2)
                src_wait(b2)
                gather(b2)
                src_load(ch + 3, b)
            return carry

        lax.fori_loop(0, (NCH - 5) // 6, six, 0)
        bq = (NCH - 3) % 3
        gather_wait(bq)
        scatter(NCH - 3, bq)
        scatter_wait(NCH - 4, (bq + 2) % 3)
        src_wait((bq + 2) % 3)
        gather((bq + 2) % 3)
        for ch in (NCH - 2, NCH - 1):
            b = ch % 3
            gather_wait(b)
            scatter(ch, b)
            scatter_wait(ch - 1, (b + 2) % 3)
        scatter_wait(NCH - 1, (NCH - 1) % 3)

        plsc.subcore_barrier()
        pltpu.sync_copy(acc_sh.at[pl.ds(row0, RPT)],
                        acc_out.at[r, c, pl.ds(row0, RPT)])
        pltpu.sync_copy(cnt_sh.at[pl.ds(row0, RPT)],
                        cnt_out.at[r, c, pl.ds(row0, RPT)])
    plsc.subcore_barrier()


def _combine_body(acc, cnt, z, o):
    out = z[...]
    for r in range(R):
        tot = acc[r, 0] + acc[r, 1]
        cr = cnt[r, 0, :, 0:1] + cnt[r, 1, :, 0:1]
        out = out + tot / jnp.maximum(cr, 1.0)
    o[...] = jnp.maximum(out, 0.0)


def _combine(acc, cnt, z):
    return pl.pallas_call(
        _combine_body,
        grid=(N // BLK,),
        in_specs=[pl.BlockSpec((R, 2, BLK, D), lambda b: (0, 0, b, 0)),
                  pl.BlockSpec((R, 2, BLK, CW), lambda b: (0, 0, b, 0)),
                  pl.BlockSpec((BLK, D), lambda b: (b, 0))],
        out_specs=pl.BlockSpec((BLK, D), lambda b: (b, 0)),
        out_shape=jax.ShapeDtypeStruct((N, D), jnp.float32),
    )(acc, cnt, z)


def kernel(x_station, x_machine, x_robot, x_job,
           edge_can_load, edge_loaded, edge_will_execute, edge_execute,
           edge_hold,
           Wl_can_load, bl_can_load, Wr_can_load,
           Wl_loaded, bl_loaded, Wr_loaded,
           Wl_will_execute, bl_will_execute, Wr_will_execute,
           Wl_execute, bl_execute, Wr_execute,
           Wl_hold, bl_hold, Wr_hold):
    edges = (edge_can_load, edge_loaded, edge_will_execute, edge_execute,
             edge_hold)
    ei = [e.astype(jnp.int32) for e in edges]
    wls = (Wl_can_load, Wl_loaded, Wl_will_execute, Wl_execute, Wl_hold)
    wrs = (Wr_can_load, Wr_loaded, Wr_will_execute, Wr_execute, Wr_hold)
    bls = (bl_can_load, bl_loaded, bl_will_execute, bl_execute, bl_hold)

    t0, t1, t2, t3, t4, z = _build_tables(x_station, x_machine, x_robot,
                                          x_job, wls, wrs, bls)
    zrows = jnp.zeros((RPT, D), jnp.float32)
    zcnt = jnp.zeros((RPT, CW), jnp.float32)
    ones_h = jnp.ones((K, CW), jnp.float32)
    acc, cnt = _sc_segment_sums(*ei, t0, t1, t2, t3, t4, zrows, zcnt, ones_h)
    return _combine(acc, cnt, z)


# speedup vs baseline: 12.9627x; 1.0403x over previous
import functools

import jax
import jax.numpy as jnp
from jax import lax
from jax.experimental import pallas as pl
from jax.experimental.pallas import tpu as pltpu
from jax.experimental.pallas import tpu_sc as plsc

N = 10000
E = 320000
D = 128
CW = 8
R = 5
BLK = 1000
NW = 32
EPT = E // NW
K = 80
NCH = EPT // K
RPT = N // 16

_DN = (((1,), (1,)), ((), ()))


def _tables_body(xst, xma, xro, xjb,
                 wl0, wl1, wl2, wl3, wl4,
                 wr0, wr1, wr2, wr3, wr4,
                 bl0, bl1, bl2, bl3, bl4,
                 t0, t1, t2, t3, t4, z):
    srcs = (xst, xst, xma, xma, xro)
    wls = (wl0, wl1, wl2, wl3, wl4)
    outs = (t0, t1, t2, t3, t4)
    for r in range(R):
        outs[r][...] = lax.dot_general(srcs[r][...], wls[r][...], _DN,
                                       preferred_element_type=jnp.float32)
    wrs = wr0[...] + wr1[...] + wr2[...] + wr3[...] + wr4[...]
    bls = bl0[...] + bl1[...] + bl2[...] + bl3[...] + bl4[...]
    z[...] = lax.dot_general(xjb[...], wrs, _DN,
                             preferred_element_type=jnp.float32) + bls[None, :]


def _build_tables(xst, xma, xro, xjb, wls, wrs, bls):
    row = pl.BlockSpec((BLK, D), lambda b: (b, 0))
    mat = pl.BlockSpec((D, D), lambda b: (0, 0))
    vec = pl.BlockSpec((D,), lambda b: (0,))
    return pl.pallas_call(
        _tables_body,
        grid=(N // BLK,),
        in_specs=[row, row, row, row] + [mat] * R + [mat] * R + [vec] * R,
        out_specs=[pl.BlockSpec((BLK, D), lambda b: (b, 0))] * (R + 1),
        out_shape=[jax.ShapeDtypeStruct((N, D), jnp.float32)] * (R + 1),
    )(xst, xma, xro, xjb, *wls, *wrs, *bls)


@functools.partial(
    pl.kernel,
    out_type=[jax.ShapeDtypeStruct((R, 2, N, D), jnp.float32),
              jax.ShapeDtypeStruct((R, 2, N, CW), jnp.float32)],
    mesh=plsc.VectorSubcoreMesh(core_axis_name="c", subcore_axis_name="s"),
    compiler_params=pltpu.CompilerParams(use_tc_tiling_on_sc=False),
    scratch_types=[
        pltpu.VMEM_SHARED((N, D), jnp.float32),
        pltpu.VMEM_SHARED((N, CW), jnp.float32),
        pltpu.VMEM((K,), jnp.int32),
        pltpu.VMEM((K,), jnp.int32),
        pltpu.VMEM((K,), jnp.int32),
        pltpu.VMEM((EPT,), jnp.int32),
        pltpu.VMEM((K, CW), jnp.float32),
        pltpu.VMEM((K, D), jnp.float32),
        pltpu.VMEM((K, D), jnp.float32),
        pltpu.VMEM((K, D), jnp.float32),
        pltpu.SemaphoreType.DMA,
        pltpu.SemaphoreType.DMA,
        pltpu.SemaphoreType.DMA,
        pltpu.SemaphoreType.DMA,
        pltpu.SemaphoreType.DMA,
        pltpu.SemaphoreType.DMA,
        pltpu.SemaphoreType.DMA,
        pltpu.SemaphoreType.DMA,
        pltpu.SemaphoreType.DMA,
        pltpu.SemaphoreType.DMA,
        pltpu.SemaphoreType.DMA,
        pltpu.SemaphoreType.DMA,
        pltpu.SemaphoreType.DMA,
        pltpu.SemaphoreType.DMA,
    ],
)
def _sc_segment_sums(e0, e1, e2, e3, e4,
                     t0, t1, t2, t3, t4,
                     zrows, zcnt, ones_h, acc_out, cnt_out,
                     acc_sh, cnt_sh, sb0, sb1, sb2, dstbuf, onesbuf,
                     rows0, rows1, rows2,
                     g0, g1, g2, w0, w1, w2, c0, c1, c2, i0, i1, i2,
                     dsem, pb):
    c = lax.axis_index("c")
    s = lax.axis_index("s")
    wid = s * 2 + c
    row0 = s * RPT
    tabs = (t0, t1, t2, t3, t4)
    eds = (e0, e1, e2, e3, e4)
    srcb = (sb0, sb1, sb2)
    rows = (rows0, rows1, rows2)
    gsem = (g0, g1, g2)
    ssem = (w0, w1, w2)
    csem = (c0, c1, c2)
    isem = (i0, i1, i2)

    pltpu.sync_copy(ones_h, onesbuf)

    aslc = acc_sh.at[pl.ds(row0, RPT)]
    cslc = cnt_sh.at[pl.ds(row0, RPT)]

    for r in range(R):
        tab = tabs[r]
        ed = eds[r]
        ebase = wid * EPT
        pltpu.async_copy(ed.at[1, pl.ds(ebase, EPT)], dstbuf, dsem)

        def src_load(ch, b):
            pltpu.async_copy(ed.at[0, pl.ds(ebase + ch * K, K)], srcb[b],
                             isem[b])

        def src_wait(b):
            pltpu.make_async_copy(ed.at[0, pl.ds(ebase, K)], srcb[b],
                                  isem[b]).wait()

        def gather(b):
            pltpu.async_copy(tab.at[srcb[b]], rows[b], gsem[b])

        def gather_wait(b):
            pltpu.make_async_copy(tab.at[srcb[b]], rows[b], gsem[b]).wait()

        def scatter(ch, b):
            idx = dstbuf.at[pl.ds(ch * K, K)]
            pltpu.async_copy(rows[b], acc_sh.at[idx], ssem[b], add=True)
            pltpu.async_copy(onesbuf, cnt_sh.at[idx], csem[b], add=True)

        def scatter_wait(ch, b):
            idx = dstbuf.at[pl.ds(ch * K, K)]
            pltpu.make_async_copy(rows[b], acc_sh.at[idx], ssem[b]).wait()
            pltpu.make_async_copy(onesbuf, cnt_sh.at[idx], csem[b]).wait()

        pltpu.sync_copy(ed.at[0, pl.ds(ebase, K)], sb0)
        pltpu.sync_copy(ed.at[0, pl.ds(ebase + K, K)], sb1)
        src_load(2, 2)
        gather(0)
        gather(1)
        if r > 0:
            pltpu.make_async_copy(aslc, acc_out.at[r - 1, c,
                                                   pl.ds(row0, RPT)],
                                  pb).wait()
            pltpu.make_async_copy(cslc, cnt_out.at[r - 1, c,
                                                   pl.ds(row0, RPT)],
                                  pb).wait()
        pltpu.async_copy(zrows, aslc, dsem)
        pltpu.async_copy(zcnt, cslc, dsem)
        pltpu.make_async_copy(ed.at[1, pl.ds(ebase, EPT)], dstbuf,
                              dsem).wait()
        pltpu.make_async_copy(zrows, aslc, dsem).wait()
        pltpu.make_async_copy(zcnt, cslc, dsem).wait()
        plsc.subcore_barrier()
        gather_wait(0)
        scatter(0, 0)
        src_wait(2)
        gather(2)
        src_load(3, 0)
        gather_wait(1)
        scatter(1, 1)
        scatter_wait(0, 0)
        src_wait(0)
        gather(0)
        src_load(4, 1)

        def six(p, carry):
            for u in range(6):
                ch = 2 + 6 * p + u
                b = (2 + u) % 3
                b2 = (b + 2) % 3
                gather_wait(b)
                scatter(ch, b)
                scatter_wait(ch - 1, b2)
                src_wait(b2)
                gather(b2)
                src_load(ch + 3, b)
            return carry

        lax.fori_loop(0, (NCH - 5) // 6, six, 0)
        bq = (NCH - 3) % 3
        gather_wait(bq)
        scatter(NCH - 3, bq)
        scatter_wait(NCH - 4, (bq + 2) % 3)
        src_wait((bq + 2) % 3)
        gather((bq + 2) % 3)
        for ch in (NCH - 2, NCH - 1):
            b = ch % 3
            gather_wait(b)
            scatter(ch, b)
            scatter_wait(ch - 1, (b + 2) % 3)
        scatter_wait(NCH - 1, (NCH - 1) % 3)

        plsc.subcore_barrier()
        pltpu.async_copy(aslc, acc_out.at[r, c, pl.ds(row0, RPT)], pb)
        pltpu.async_copy(cslc, cnt_out.at[r, c, pl.ds(row0, RPT)], pb)
    pltpu.make_async_copy(aslc, acc_out.at[R - 1, c, pl.ds(row0, RPT)],
                          pb).wait()
    pltpu.make_async_copy(cslc, cnt_out.at[R - 1, c, pl.ds(row0, RPT)],
                          pb).wait()
    plsc.subcore_barrier()


def _combine_body(acc, cnt, z, o):
    out = z[...]
    for r in range(R):
        tot = acc[r, 0] + acc[r, 1]
        cr = cnt[r, 0, :, 0:1] + cnt[r, 1, :, 0:1]
        out = out + tot / jnp.maximum(cr, 1.0)
    o[...] = jnp.maximum(out, 0.0)


def _combine(acc, cnt, z):
    return pl.pallas_call(
        _combine_body,
        grid=(N // BLK,),
        in_specs=[pl.BlockSpec((R, 2, BLK, D), lambda b: (0, 0, b, 0)),
                  pl.BlockSpec((R, 2, BLK, CW), lambda b: (0, 0, b, 0)),
                  pl.BlockSpec((BLK, D), lambda b: (b, 0))],
        out_specs=pl.BlockSpec((BLK, D), lambda b: (b, 0)),
        out_shape=jax.ShapeDtypeStruct((N, D), jnp.float32),
    )(acc, cnt, z)


def kernel(x_station, x_machine, x_robot, x_job,
           edge_can_load, edge_loaded, edge_will_execute, edge_execute,
           edge_hold,
           Wl_can_load, bl_can_load, Wr_can_load,
           Wl_loaded, bl_loaded, Wr_loaded,
           Wl_will_execute, bl_will_execute, Wr_will_execute,
           Wl_execute, bl_execute, Wr_execute,
           Wl_hold, bl_hold, Wr_hold):
    edges = (edge_can_load, edge_loaded, edge_will_execute, edge_execute,
             edge_hold)
    ei = [e.astype(jnp.int32) for e in edges]
    wls = (Wl_can_load, Wl_loaded, Wl_will_execute, Wl_execute, Wl_hold)
    wrs = (Wr_can_load, Wr_loaded, Wr_will_execute, Wr_execute, Wr_hold)
    bls = (bl_can_load, bl_loaded, bl_will_execute, bl_execute, bl_hold)

    t0, t1, t2, t3, t4, z = _build_tables(x_station, x_machine, x_robot,
                                          x_job, wls, wrs, bls)
    zrows = jnp.zeros((RPT, D), jnp.float32)
    zcnt = jnp.zeros((RPT, CW), jnp.float32)
    ones_h = jnp.ones((K, CW), jnp.float32)
    acc, cnt = _sc_segment_sums(*ei, t0, t1, t2, t3, t4, zrows, zcnt, ones_h)
    return _combine(acc, cnt, z)
